# baseline (speedup 1.0000x reference)
import functools

import jax
import jax.numpy as jnp
from jax import lax
from jax.experimental import pallas as pl
from jax.experimental.pallas import tpu as pltpu

_N = 20000
_PAD = 20480
_K = 1024
_MAX_OUT = 500
_KSLOT = 4
_LANES = 128
_THR = 0.5


def _nms_body(nlimit, rows_ref, out_ref, flag_ref,
              kx1_ref, ky1_ref, kx2_ref, ky2_ref, karea_ref):
    out_ref[...] = jnp.zeros((_MAX_OUT, 5), jnp.float32)
    kx1_ref[...] = jnp.zeros((_KSLOT, _LANES), jnp.float32)
    ky1_ref[...] = jnp.zeros((_KSLOT, _LANES), jnp.float32)
    kx2_ref[...] = jnp.zeros((_KSLOT, _LANES), jnp.float32)
    ky2_ref[...] = jnp.zeros((_KSLOT, _LANES), jnp.float32)
    karea_ref[...] = jnp.zeros((_KSLOT, _LANES), jnp.float32)

    slot_rows = lax.broadcasted_iota(jnp.int32, (_KSLOT, _LANES), 0)
    slot_lanes = lax.broadcasted_iota(jnp.int32, (_KSLOT, _LANES), 1)
    slot_iota = slot_rows * _LANES + slot_lanes

    def cond(state):
        p, count = state
        return jnp.logical_and(count < _MAX_OUT, p < nlimit)

    def body(state):
        p, count = state
        b = rows_ref[pl.ds(p, 1), :]
        bx1 = b[:, 0:1]
        by1 = b[:, 1:2]
        bx2 = b[:, 2:3]
        by2 = b[:, 3:4]
        bs = b[:, 4:5]

        xx1 = jnp.maximum(kx1_ref[...], bx1)
        yy1 = jnp.maximum(ky1_ref[...], by1)
        xx2 = jnp.minimum(kx2_ref[...], bx2)
        yy2 = jnp.minimum(ky2_ref[...], by2)
        w = jnp.maximum(xx2 - xx1, 0.0)
        h = jnp.maximum(yy2 - yy1, 0.0)
        inter = w * h
        area_a = (bx2 - bx1) * (by2 - by1)
        iou = inter / (area_a + karea_ref[...] - inter + 1e-6)
        hit = jnp.logical_and(iou >= _THR, slot_iota < count)
        keep = jnp.logical_not(jnp.any(hit))

        @pl.when(keep)
        def _():
            onehot = slot_iota == count
            kx1_ref[...] = jnp.where(onehot, bx1, kx1_ref[...])
            ky1_ref[...] = jnp.where(onehot, by1, ky1_ref[...])
            kx2_ref[...] = jnp.where(onehot, bx2, kx2_ref[...])
            ky2_ref[...] = jnp.where(onehot, by2, ky2_ref[...])
            karea_ref[...] = jnp.where(onehot, area_a, karea_ref[...])
            out_ref[pl.ds(count, 1), :] = b

        return (p + 1, count + keep.astype(jnp.int32))

    _, count = lax.while_loop(cond, body, (jnp.int32(0), jnp.int32(0)))
    flag_ref[...] = jnp.reshape((count < _MAX_OUT).astype(jnp.int32), (1, 1))


def _run_nms(rows, nlimit):
    return pl.pallas_call(
        functools.partial(_nms_body, nlimit),
        out_shape=(
            jax.ShapeDtypeStruct((_MAX_OUT, 5), jnp.float32),
            jax.ShapeDtypeStruct((1, 1), jnp.int32),
        ),
        scratch_shapes=[pltpu.VMEM((_KSLOT, _LANES), jnp.float32)] * 5,
    )(rows)


def kernel(boxes, scores):
    ss, order = lax.top_k(scores, _K)
    sb = jnp.take(boxes, order, axis=0)
    rows = jnp.concatenate([sb, ss[:, None]], axis=1)
    out_fast, flag = _run_nms(rows, _K)

    def full_path(_):
        order_f = jnp.argsort(-scores)
        sb_f = jnp.take(boxes, order_f, axis=0)
        ss_f = jnp.take(scores, order_f, axis=0)
        rows_f = jnp.concatenate([sb_f, ss_f[:, None]], axis=1)
        rows_f = jnp.pad(rows_f, ((0, _PAD - _N), (0, 0)))
        out_full, _unused = _run_nms(rows_f, _N)
        return out_full

    return lax.cond(flag[0, 0] > 0, full_path, lambda _: out_fast, None)


# speedup vs baseline: 1.4679x; 1.4679x over previous
import functools

import jax
import jax.numpy as jnp
from jax import lax
from jax.experimental import pallas as pl
from jax.experimental.pallas import tpu as pltpu

_N = 20000
_PAD = 20480
_K = 1024
_MAX_OUT = 500
_KSLOT = 4
_LANES = 128
_THR = 0.5


def _nms_body(nlimit, planes_ref, out_ref, flag_ref,
              kx1_ref, ky1_ref, kx2_ref, ky2_ref, karea_ref):
    out_ref[...] = jnp.zeros((_MAX_OUT, 5), jnp.float32)
    kx1_ref[...] = jnp.zeros((_KSLOT, _LANES), jnp.float32)
    ky1_ref[...] = jnp.zeros((_KSLOT, _LANES), jnp.float32)
    kx2_ref[...] = jnp.zeros((_KSLOT, _LANES), jnp.float32)
    ky2_ref[...] = jnp.zeros((_KSLOT, _LANES), jnp.float32)
    karea_ref[...] = jnp.zeros((_KSLOT, _LANES), jnp.float32)

    lane_iota = lax.broadcasted_iota(jnp.int32, (1, 1, _LANES), 2)
    slot_rows = lax.broadcasted_iota(jnp.int32, (_KSLOT, _LANES), 0)
    slot_lanes = lax.broadcasted_iota(jnp.int32, (_KSLOT, _LANES), 1)
    slot_iota = slot_rows * _LANES + slot_lanes

    def cond(state):
        p, count = state
        return jnp.logical_and(count < _MAX_OUT, p < nlimit)

    def body(state):
        p, count = state
        r = p // _LANES
        c = p - r * _LANES
        blk = planes_ref[:, pl.ds(r, 1), :]
        sel = jnp.sum(jnp.where(lane_iota == c, blk, 0.0), axis=2)
        bx1 = sel[0:1, :]
        by1 = sel[1:2, :]
        bx2 = sel[2:3, :]
        by2 = sel[3:4, :]
        bs = sel[4:5, :]

        xx1 = jnp.maximum(kx1_ref[...], bx1)
        yy1 = jnp.maximum(ky1_ref[...], by1)
        xx2 = jnp.minimum(kx2_ref[...], bx2)
        yy2 = jnp.minimum(ky2_ref[...], by2)
        w = jnp.maximum(xx2 - xx1, 0.0)
        h = jnp.maximum(yy2 - yy1, 0.0)
        inter = w * h
        area_a = (bx2 - bx1) * (by2 - by1)
        iou = inter / (area_a + karea_ref[...] - inter + 1e-6)
        hit = jnp.logical_and(iou >= _THR, slot_iota < count)
        keep = jnp.logical_not(jnp.any(hit))

        @pl.when(keep)
        def _():
            onehot = slot_iota == count
            kx1_ref[...] = jnp.where(onehot, bx1, kx1_ref[...])
            ky1_ref[...] = jnp.where(onehot, by1, ky1_ref[...])
            kx2_ref[...] = jnp.where(onehot, bx2, kx2_ref[...])
            ky2_ref[...] = jnp.where(onehot, by2, ky2_ref[...])
            karea_ref[...] = jnp.where(onehot, area_a, karea_ref[...])
            out_ref[pl.ds(count, 1), 0:1] = bx1
            out_ref[pl.ds(count, 1), 1:2] = by1
            out_ref[pl.ds(count, 1), 2:3] = bx2
            out_ref[pl.ds(count, 1), 3:4] = by2
            out_ref[pl.ds(count, 1), 4:5] = bs

        return (p + 1, count + keep.astype(jnp.int32))

    _, count = lax.while_loop(cond, body, (jnp.int32(0), jnp.int32(0)))
    flag_ref[...] = jnp.reshape((count < _MAX_OUT).astype(jnp.int32), (1, 1))


def _run_nms(planes, nlimit):
    return pl.pallas_call(
        functools.partial(_nms_body, nlimit),
        out_shape=(
            jax.ShapeDtypeStruct((_MAX_OUT, 5), jnp.float32),
            jax.ShapeDtypeStruct((1, 1), jnp.int32),
        ),
        scratch_shapes=[pltpu.VMEM((_KSLOT, _LANES), jnp.float32)] * 5,
    )(planes)


def _make_planes(sb, ss, npad):
    cols = jnp.concatenate([sb, ss[:, None]], axis=1)
    cols = jnp.pad(cols, ((0, npad - cols.shape[0]), (0, 0)))
    return cols.T.reshape(5, npad // _LANES, _LANES)


def kernel(boxes, scores):
    ss, order = lax.top_k(scores, _K)
    sb = jnp.take(boxes, order, axis=0)
    out_fast, flag = _run_nms(_make_planes(sb, ss, _K), _K)

    def full_path(_):
        order_f = jnp.argsort(-scores)
        sb_f = jnp.take(boxes, order_f, axis=0)
        ss_f = jnp.take(scores, order_f, axis=0)
        out_full, _unused = _run_nms(_make_planes(sb_f, ss_f, _PAD), _N)
        return out_full

    return lax.cond(flag[0, 0] > 0, full_path, lambda _: out_fast, None)


# speedup vs baseline: 4.2655x; 2.9058x over previous
import functools

import jax
import jax.numpy as jnp
from jax import lax
from jax.experimental import pallas as pl
from jax.experimental.pallas import tpu as pltpu

_N = 20000
_PAD = 20480
_K = 1024
_NCHUNK = _K // 128
_MAX_OUT = 500
_KSLOT = 4
_SLOTS = 640
_LANES = 128
_THR = 0.5


def _nms_chunked_body(planes_ref, sub_ref, out_ref, flag_ref,
                      kx1_ref, ky1_ref, kx2_ref, ky2_ref, karea_ref,
                      oacc_ref, kl_ref):
    out_ref[...] = jnp.zeros((_MAX_OUT, 5), jnp.float32)
    kx1_ref[...] = jnp.zeros((_SLOTS, _LANES), jnp.float32)
    ky1_ref[...] = jnp.zeros((_SLOTS, _LANES), jnp.float32)
    kx2_ref[...] = jnp.zeros((_SLOTS, _LANES), jnp.float32)
    ky2_ref[...] = jnp.zeros((_SLOTS, _LANES), jnp.float32)
    karea_ref[...] = jnp.zeros((_SLOTS, _LANES), jnp.float32)
    oacc_ref[...] = jnp.zeros((_SLOTS, 8), jnp.float32)

    slotS = lax.broadcasted_iota(jnp.int32, (_SLOTS, 1), 0)
    sub2 = lax.broadcasted_iota(jnp.int32, (_LANES, _LANES), 0)
    lane2 = lax.broadcasted_iota(jnp.int32, (_LANES, _LANES), 1)
    ltri = (sub2 < lane2).astype(jnp.float32)

    def cond(state):
        cidx, count = state
        return jnp.logical_and(cidx < _NCHUNK, count < _MAX_OUT)

    def body(state):
        cidx, count = state

        def getL(i):
            return jnp.reshape(planes_ref[i:i + 1, pl.ds(cidx, 1), :],
                               (1, _LANES))

        def getS(i):
            return jnp.reshape(sub_ref[i:i + 1, pl.ds(cidx * _LANES, _LANES), :],
                               (_LANES, 1))

        bx1L, by1L, bx2L, by2L = getL(0), getL(1), getL(2), getL(3)
        bx1S, by1S, bx2S, by2S, bsS = getS(0), getS(1), getS(2), getS(3), getS(4)
        areaL = (bx2L - bx1L) * (by2L - by1L)
        areaS = (bx2S - bx1S) * (by2S - by1S)

        xx1 = jnp.maximum(kx1_ref[...], bx1L)
        yy1 = jnp.maximum(ky1_ref[...], by1L)
        xx2 = jnp.minimum(kx2_ref[...], bx2L)
        yy2 = jnp.minimum(ky2_ref[...], by2L)
        w = jnp.maximum(xx2 - xx1, 0.0)
        h = jnp.maximum(yy2 - yy1, 0.0)
        inter = w * h
        iou = inter / (areaL + karea_ref[...] - inter + 1e-6)
        hit = jnp.logical_and(iou >= _THR, slotS < count)
        alive0 = jnp.logical_not(jnp.any(hit, axis=0, keepdims=True))

        mx1 = jnp.maximum(bx1S, bx1L)
        my1 = jnp.maximum(by1S, by1L)
        mx2 = jnp.minimum(bx2S, bx2L)
        my2 = jnp.minimum(by2S, by2L)
        mw = jnp.maximum(mx2 - mx1, 0.0)
        mh = jnp.maximum(my2 - my1, 0.0)
        minter = mw * mh
        miou = minter / (areaS + areaL - minter + 1e-6)
        mhit = jnp.logical_and(miou >= _THR, sub2 < lane2)

        kl_ref[...] = alive0.astype(jnp.int32)

        def fix_body(_):
            kl = kl_ref[...] != 0
            ks = jnp.any(jnp.logical_and(lane2 == sub2, kl), axis=1,
                         keepdims=True)
            sup = jnp.any(jnp.logical_and(mhit, ks), axis=0,
                          keepdims=True)
            knew = jnp.logical_and(alive0, jnp.logical_not(sup))
            kl_ref[...] = knew.astype(jnp.int32)
            return jnp.any(knew != kl)

        lax.while_loop(lambda c: c, fix_body, True)
        keepL = kl_ref[...] != 0
        keepf = keepL.astype(jnp.float32)

        prefixL = lax.dot_general(keepf, ltri, (((1,), (0,)), ((), ())),
                                  precision=lax.Precision.HIGHEST)
        pmat = jnp.logical_and(sub2 == prefixL.astype(jnp.int32),
                               keepL).astype(jnp.float32)
        vmat = jnp.concatenate([bx1S, by1S, bx2S, by2S, bsS, areaS,
                                jnp.zeros((_LANES, 2), jnp.float32)],
                               axis=1)
        compact = lax.dot_general(pmat, vmat, (((1,), (0,)), ((), ())),
                                  precision=lax.Precision.HIGHEST)

        oacc_ref[pl.ds(count, _LANES), :] = compact
        kx1_ref[pl.ds(count, _LANES), :] = jnp.broadcast_to(
            compact[:, 0:1], (_LANES, _LANES))
        ky1_ref[pl.ds(count, _LANES), :] = jnp.broadcast_to(
            compact[:, 1:2], (_LANES, _LANES))
        kx2_ref[pl.ds(count, _LANES), :] = jnp.broadcast_to(
            compact[:, 2:3], (_LANES, _LANES))
        ky2_ref[pl.ds(count, _LANES), :] = jnp.broadcast_to(
            compact[:, 3:4], (_LANES, _LANES))
        karea_ref[pl.ds(count, _LANES), :] = jnp.broadcast_to(
            compact[:, 5:6], (_LANES, _LANES))

        nkeep = jnp.sum(keepf).astype(jnp.int32)
        return (cidx + 1, count + nkeep)

    _, count = lax.while_loop(cond, body, (jnp.int32(0), jnp.int32(0)))
    out_ref[...] = oacc_ref[0:_MAX_OUT, 0:5]
    flag_ref[...] = jnp.reshape((count < _MAX_OUT).astype(jnp.int32), (1, 1))


def _run_nms_chunked(planes, subplanes):
    return pl.pallas_call(
        _nms_chunked_body,
        out_shape=(
            jax.ShapeDtypeStruct((_MAX_OUT, 5), jnp.float32),
            jax.ShapeDtypeStruct((1, 1), jnp.int32),
        ),
        scratch_shapes=[pltpu.VMEM((_SLOTS, _LANES), jnp.float32)] * 5
        + [pltpu.VMEM((_SLOTS, 8), jnp.float32),
           pltpu.VMEM((1, _LANES), jnp.int32)],
    )(planes, subplanes)


def _nms_body(nlimit, planes_ref, out_ref, flag_ref,
              kx1_ref, ky1_ref, kx2_ref, ky2_ref, karea_ref):
    out_ref[...] = jnp.zeros((_MAX_OUT, 5), jnp.float32)
    kx1_ref[...] = jnp.zeros((_KSLOT, _LANES), jnp.float32)
    ky1_ref[...] = jnp.zeros((_KSLOT, _LANES), jnp.float32)
    kx2_ref[...] = jnp.zeros((_KSLOT, _LANES), jnp.float32)
    ky2_ref[...] = jnp.zeros((_KSLOT, _LANES), jnp.float32)
    karea_ref[...] = jnp.zeros((_KSLOT, _LANES), jnp.float32)

    lane_iota = lax.broadcasted_iota(jnp.int32, (1, 1, _LANES), 2)
    slot_rows = lax.broadcasted_iota(jnp.int32, (_KSLOT, _LANES), 0)
    slot_lanes = lax.broadcasted_iota(jnp.int32, (_KSLOT, _LANES), 1)
    slot_iota = slot_rows * _LANES + slot_lanes

    def cond(state):
        p, count = state
        return jnp.logical_and(count < _MAX_OUT, p < nlimit)

    def body(state):
        p, count = state
        r = p // _LANES
        c = p - r * _LANES
        blk = planes_ref[:, pl.ds(r, 1), :]
        sel = jnp.sum(jnp.where(lane_iota == c, blk, 0.0), axis=2)
        bx1 = sel[0:1, :]
        by1 = sel[1:2, :]
        bx2 = sel[2:3, :]
        by2 = sel[3:4, :]
        bs = sel[4:5, :]

        xx1 = jnp.maximum(kx1_ref[...], bx1)
        yy1 = jnp.maximum(ky1_ref[...], by1)
        xx2 = jnp.minimum(kx2_ref[...], bx2)
        yy2 = jnp.minimum(ky2_ref[...], by2)
        w = jnp.maximum(xx2 - xx1, 0.0)
        h = jnp.maximum(yy2 - yy1, 0.0)
        inter = w * h
        area_a = (bx2 - bx1) * (by2 - by1)
        iou = inter / (area_a + karea_ref[...] - inter + 1e-6)
        hit = jnp.logical_and(iou >= _THR, slot_iota < count)
        keep = jnp.logical_not(jnp.any(hit))

        @pl.when(keep)
        def _():
            onehot = slot_iota == count
            kx1_ref[...] = jnp.where(onehot, bx1, kx1_ref[...])
            ky1_ref[...] = jnp.where(onehot, by1, ky1_ref[...])
            kx2_ref[...] = jnp.where(onehot, bx2, kx2_ref[...])
            ky2_ref[...] = jnp.where(onehot, by2, ky2_ref[...])
            karea_ref[...] = jnp.where(onehot, area_a, karea_ref[...])
            out_ref[pl.ds(count, 1), 0:1] = bx1
            out_ref[pl.ds(count, 1), 1:2] = by1
            out_ref[pl.ds(count, 1), 2:3] = bx2
            out_ref[pl.ds(count, 1), 3:4] = by2
            out_ref[pl.ds(count, 1), 4:5] = bs

        return (p + 1, count + keep.astype(jnp.int32))

    _, count = lax.while_loop(cond, body, (jnp.int32(0), jnp.int32(0)))
    flag_ref[...] = jnp.reshape((count < _MAX_OUT).astype(jnp.int32), (1, 1))


def _run_nms(planes, nlimit):
    return pl.pallas_call(
        functools.partial(_nms_body, nlimit),
        out_shape=(
            jax.ShapeDtypeStruct((_MAX_OUT, 5), jnp.float32),
            jax.ShapeDtypeStruct((1, 1), jnp.int32),
        ),
        scratch_shapes=[pltpu.VMEM((_KSLOT, _LANES), jnp.float32)] * 5,
    )(planes)


def _make_planes(sb, ss, npad):
    cols = jnp.concatenate([sb, ss[:, None]], axis=1)
    cols = jnp.pad(cols, ((0, npad - cols.shape[0]), (0, 0)))
    return cols.T.reshape(5, npad // _LANES, _LANES)


def kernel(boxes, scores):
    ss, order = lax.top_k(scores, _K)
    sb = jnp.take(boxes, order, axis=0)
    planes = _make_planes(sb, ss, _K)
    subplanes = jnp.concatenate([sb, ss[:, None]], axis=1).T.reshape(5, _K, 1)
    out_fast, flag = _run_nms_chunked(planes, subplanes)

    def full_path(_):
        order_f = jnp.argsort(-scores)
        sb_f = jnp.take(boxes, order_f, axis=0)
        ss_f = jnp.take(scores, order_f, axis=0)
        out_full, _unused = _run_nms(_make_planes(sb_f, ss_f, _PAD), _N)
        return out_full

    return lax.cond(flag[0, 0] > 0, full_path, lambda _: out_fast, None)


# speedup vs baseline: 4.4285x; 1.0382x over previous
import functools

import jax
import jax.numpy as jnp
from jax import lax
from jax.experimental import pallas as pl
from jax.experimental.pallas import tpu as pltpu

_N = 20000
_PAD = 20480
_K = 640
_NCHUNK = _K // 128
_MAX_OUT = 500
_KSLOT = 4
_SLOTS = 640
_LANES = 128
_THR = 0.5


def _nms_chunked_body(planes_ref, sub_ref, out_ref, flag_ref,
                      kx1_ref, ky1_ref, kx2_ref, ky2_ref, karea_ref,
                      oacc_ref, kl_ref):
    out_ref[...] = jnp.zeros((_MAX_OUT, 5), jnp.float32)
    kx1_ref[...] = jnp.zeros((_SLOTS, _LANES), jnp.float32)
    ky1_ref[...] = jnp.zeros((_SLOTS, _LANES), jnp.float32)
    kx2_ref[...] = jnp.zeros((_SLOTS, _LANES), jnp.float32)
    ky2_ref[...] = jnp.zeros((_SLOTS, _LANES), jnp.float32)
    karea_ref[...] = jnp.zeros((_SLOTS, _LANES), jnp.float32)
    oacc_ref[...] = jnp.zeros((_SLOTS, 8), jnp.float32)

    slotS = lax.broadcasted_iota(jnp.int32, (_SLOTS, 1), 0)
    sub2 = lax.broadcasted_iota(jnp.int32, (_LANES, _LANES), 0)
    lane2 = lax.broadcasted_iota(jnp.int32, (_LANES, _LANES), 1)
    ltri = (sub2 < lane2).astype(jnp.float32)

    def cond(state):
        cidx, count = state
        return jnp.logical_and(cidx < _NCHUNK, count < _MAX_OUT)

    def body(state):
        cidx, count = state

        def getL(i):
            return jnp.reshape(planes_ref[i:i + 1, pl.ds(cidx, 1), :],
                               (1, _LANES))

        def getS(i):
            return jnp.reshape(sub_ref[i:i + 1, pl.ds(cidx * _LANES, _LANES), :],
                               (_LANES, 1))

        bx1L, by1L, bx2L, by2L = getL(0), getL(1), getL(2), getL(3)
        bx1S, by1S, bx2S, by2S, bsS = getS(0), getS(1), getS(2), getS(3), getS(4)
        areaL = (bx2L - bx1L) * (by2L - by1L)
        areaS = (bx2S - bx1S) * (by2S - by1S)

        xx1 = jnp.maximum(kx1_ref[0:512, :], bx1L)
        yy1 = jnp.maximum(ky1_ref[0:512, :], by1L)
        xx2 = jnp.minimum(kx2_ref[0:512, :], bx2L)
        yy2 = jnp.minimum(ky2_ref[0:512, :], by2L)
        w = jnp.maximum(xx2 - xx1, 0.0)
        h = jnp.maximum(yy2 - yy1, 0.0)
        inter = w * h
        iou = inter / (areaL + karea_ref[0:512, :] - inter + 1e-6)
        hit = jnp.logical_and(iou >= _THR, slotS[0:512, :] < count)
        alive0 = jnp.logical_not(jnp.any(hit, axis=0, keepdims=True))

        mx1 = jnp.maximum(bx1S, bx1L)
        my1 = jnp.maximum(by1S, by1L)
        mx2 = jnp.minimum(bx2S, bx2L)
        my2 = jnp.minimum(by2S, by2L)
        mw = jnp.maximum(mx2 - mx1, 0.0)
        mh = jnp.maximum(my2 - my1, 0.0)
        minter = mw * mh
        miou = minter / (areaS + areaL - minter + 1e-6)
        mhit = jnp.logical_and(miou >= _THR, sub2 < lane2)

        kl_ref[...] = alive0.astype(jnp.int32)

        def fix_body(_):
            kl = kl_ref[...] != 0
            ks = jnp.any(jnp.logical_and(lane2 == sub2, kl), axis=1,
                         keepdims=True)
            sup = jnp.any(jnp.logical_and(mhit, ks), axis=0,
                          keepdims=True)
            knew = jnp.logical_and(alive0, jnp.logical_not(sup))
            kl_ref[...] = knew.astype(jnp.int32)
            return jnp.any(knew != kl)

        lax.while_loop(lambda c: c, fix_body, True)
        keepL = kl_ref[...] != 0
        keepf = keepL.astype(jnp.float32)

        prefixL = lax.dot_general(keepf, ltri, (((1,), (0,)), ((), ())),
                                  precision=lax.Precision.HIGHEST)
        pmat = jnp.logical_and(sub2 == prefixL.astype(jnp.int32),
                               keepL).astype(jnp.float32)
        vmat = jnp.concatenate([bx1S, by1S, bx2S, by2S, bsS, areaS,
                                jnp.zeros((_LANES, 2), jnp.float32)],
                               axis=1)
        compact = lax.dot_general(pmat, vmat, (((1,), (0,)), ((), ())),
                                  precision=lax.Precision.HIGHEST)

        oacc_ref[pl.ds(count, _LANES), :] = compact
        kx1_ref[pl.ds(count, _LANES), :] = jnp.broadcast_to(
            compact[:, 0:1], (_LANES, _LANES))
        ky1_ref[pl.ds(count, _LANES), :] = jnp.broadcast_to(
            compact[:, 1:2], (_LANES, _LANES))
        kx2_ref[pl.ds(count, _LANES), :] = jnp.broadcast_to(
            compact[:, 2:3], (_LANES, _LANES))
        ky2_ref[pl.ds(count, _LANES), :] = jnp.broadcast_to(
            compact[:, 3:4], (_LANES, _LANES))
        karea_ref[pl.ds(count, _LANES), :] = jnp.broadcast_to(
            compact[:, 5:6], (_LANES, _LANES))

        nkeep = jnp.sum(keepf).astype(jnp.int32)
        return (cidx + 1, count + nkeep)

    _, count = lax.while_loop(cond, body, (jnp.int32(0), jnp.int32(0)))
    out_ref[...] = oacc_ref[0:_MAX_OUT, 0:5]
    flag_ref[...] = jnp.reshape((count < _MAX_OUT).astype(jnp.int32), (1, 1))


def _run_nms_chunked(planes, subplanes):
    return pl.pallas_call(
        _nms_chunked_body,
        out_shape=(
            jax.ShapeDtypeStruct((_MAX_OUT, 5), jnp.float32),
            jax.ShapeDtypeStruct((1, 1), jnp.int32),
        ),
        scratch_shapes=[pltpu.VMEM((_SLOTS, _LANES), jnp.float32)] * 5
        + [pltpu.VMEM((_SLOTS, 8), jnp.float32),
           pltpu.VMEM((1, _LANES), jnp.int32)],
    )(planes, subplanes)


def _nms_body(nlimit, planes_ref, out_ref, flag_ref,
              kx1_ref, ky1_ref, kx2_ref, ky2_ref, karea_ref):
    out_ref[...] = jnp.zeros((_MAX_OUT, 5), jnp.float32)
    kx1_ref[...] = jnp.zeros((_KSLOT, _LANES), jnp.float32)
    ky1_ref[...] = jnp.zeros((_KSLOT, _LANES), jnp.float32)
    kx2_ref[...] = jnp.zeros((_KSLOT, _LANES), jnp.float32)
    ky2_ref[...] = jnp.zeros((_KSLOT, _LANES), jnp.float32)
    karea_ref[...] = jnp.zeros((_KSLOT, _LANES), jnp.float32)

    lane_iota = lax.broadcasted_iota(jnp.int32, (1, 1, _LANES), 2)
    slot_rows = lax.broadcasted_iota(jnp.int32, (_KSLOT, _LANES), 0)
    slot_lanes = lax.broadcasted_iota(jnp.int32, (_KSLOT, _LANES), 1)
    slot_iota = slot_rows * _LANES + slot_lanes

    def cond(state):
        p, count = state
        return jnp.logical_and(count < _MAX_OUT, p < nlimit)

    def body(state):
        p, count = state
        r = p // _LANES
        c = p - r * _LANES
        blk = planes_ref[:, pl.ds(r, 1), :]
        sel = jnp.sum(jnp.where(lane_iota == c, blk, 0.0), axis=2)
        bx1 = sel[0:1, :]
        by1 = sel[1:2, :]
        bx2 = sel[2:3, :]
        by2 = sel[3:4, :]
        bs = sel[4:5, :]

        xx1 = jnp.maximum(kx1_ref[...], bx1)
        yy1 = jnp.maximum(ky1_ref[...], by1)
        xx2 = jnp.minimum(kx2_ref[...], bx2)
        yy2 = jnp.minimum(ky2_ref[...], by2)
        w = jnp.maximum(xx2 - xx1, 0.0)
        h = jnp.maximum(yy2 - yy1, 0.0)
        inter = w * h
        area_a = (bx2 - bx1) * (by2 - by1)
        iou = inter / (area_a + karea_ref[...] - inter + 1e-6)
        hit = jnp.logical_and(iou >= _THR, slot_iota < count)
        keep = jnp.logical_not(jnp.any(hit))

        @pl.when(keep)
        def _():
            onehot = slot_iota == count
            kx1_ref[...] = jnp.where(onehot, bx1, kx1_ref[...])
            ky1_ref[...] = jnp.where(onehot, by1, ky1_ref[...])
            kx2_ref[...] = jnp.where(onehot, bx2, kx2_ref[...])
            ky2_ref[...] = jnp.where(onehot, by2, ky2_ref[...])
            karea_ref[...] = jnp.where(onehot, area_a, karea_ref[...])
            out_ref[pl.ds(count, 1), 0:1] = bx1
            out_ref[pl.ds(count, 1), 1:2] = by1
            out_ref[pl.ds(count, 1), 2:3] = bx2
            out_ref[pl.ds(count, 1), 3:4] = by2
            out_ref[pl.ds(count, 1), 4:5] = bs

        return (p + 1, count + keep.astype(jnp.int32))

    _, count = lax.while_loop(cond, body, (jnp.int32(0), jnp.int32(0)))
    flag_ref[...] = jnp.reshape((count < _MAX_OUT).astype(jnp.int32), (1, 1))


def _run_nms(planes, nlimit):
    return pl.pallas_call(
        functools.partial(_nms_body, nlimit),
        out_shape=(
            jax.ShapeDtypeStruct((_MAX_OUT, 5), jnp.float32),
            jax.ShapeDtypeStruct((1, 1), jnp.int32),
        ),
        scratch_shapes=[pltpu.VMEM((_KSLOT, _LANES), jnp.float32)] * 5,
    )(planes)


def _make_planes(sb, ss, npad):
    cols = jnp.concatenate([sb, ss[:, None]], axis=1)
    cols = jnp.pad(cols, ((0, npad - cols.shape[0]), (0, 0)))
    return cols.T.reshape(5, npad // _LANES, _LANES)


def kernel(boxes, scores):
    ss, order = lax.top_k(scores, _K)
    sb = jnp.take(boxes, order, axis=0)
    planes = _make_planes(sb, ss, _K)
    subplanes = jnp.concatenate([sb, ss[:, None]], axis=1).T.reshape(5, _K, 1)
    out_fast, flag = _run_nms_chunked(planes, subplanes)

    def full_path(_):
        order_f = jnp.argsort(-scores)
        sb_f = jnp.take(boxes, order_f, axis=0)
        ss_f = jnp.take(scores, order_f, axis=0)
        out_full, _unused = _run_nms(_make_planes(sb_f, ss_f, _PAD), _N)
        return out_full

    return lax.cond(flag[0, 0] > 0, full_path, lambda _: out_fast, None)


# speedup vs baseline: 4.4988x; 1.0159x over previous
import functools

import jax
import jax.numpy as jnp
from jax import lax
from jax.experimental import pallas as pl
from jax.experimental.pallas import tpu as pltpu

_N = 20000
_PAD = 20480
_K = 640
_NCHUNK = _K // 128
_MAX_OUT = 500
_KSLOT = 4
_SLOTS = 640
_LANES = 128
_THR = 0.5


def _nms_chunked_body(planes_ref, sub_ref, out_ref, flag_ref,
                      kx1_ref, ky1_ref, kx2_ref, ky2_ref, karea_ref,
                      oacc_ref, kl_ref):
    out_ref[...] = jnp.zeros((_MAX_OUT, 5), jnp.float32)
    kx1_ref[...] = jnp.full((_SLOTS, _LANES), -1e9, jnp.float32)
    ky1_ref[...] = jnp.full((_SLOTS, _LANES), -1e9, jnp.float32)
    kx2_ref[...] = jnp.full((_SLOTS, _LANES), -1e9, jnp.float32)
    ky2_ref[...] = jnp.full((_SLOTS, _LANES), -1e9, jnp.float32)
    karea_ref[...] = jnp.zeros((_SLOTS, _LANES), jnp.float32)
    oacc_ref[...] = jnp.zeros((_SLOTS, 8), jnp.float32)

    sub2 = lax.broadcasted_iota(jnp.int32, (_LANES, _LANES), 0)
    lane2 = lax.broadcasted_iota(jnp.int32, (_LANES, _LANES), 1)
    ltri = (sub2 < lane2).astype(jnp.float32)

    def cond(state):
        cidx, count = state
        return jnp.logical_and(cidx < _NCHUNK, count < _MAX_OUT)

    def body(state):
        cidx, count = state

        def getL(i):
            return jnp.reshape(planes_ref[i:i + 1, pl.ds(cidx, 1), :],
                               (1, _LANES))

        def getS(i):
            return jnp.reshape(sub_ref[i:i + 1, pl.ds(cidx * _LANES, _LANES), :],
                               (_LANES, 1))

        bx1L, by1L, bx2L, by2L = getL(0), getL(1), getL(2), getL(3)
        bx1S, by1S, bx2S, by2S, bsS = getS(0), getS(1), getS(2), getS(3), getS(4)
        areaL = (bx2L - bx1L) * (by2L - by1L)
        areaS = (bx2S - bx1S) * (by2S - by1S)

        xx1 = jnp.maximum(kx1_ref[...], bx1L)
        yy1 = jnp.maximum(ky1_ref[...], by1L)
        xx2 = jnp.minimum(kx2_ref[...], bx2L)
        yy2 = jnp.minimum(ky2_ref[...], by2L)
        w = jnp.maximum(xx2 - xx1, 0.0)
        h = jnp.maximum(yy2 - yy1, 0.0)
        inter = w * h
        iou = inter / (areaL + karea_ref[...] - inter + 1e-6)
        hit = iou >= _THR
        alive0 = jnp.logical_not(jnp.any(hit, axis=0, keepdims=True))

        mx1 = jnp.maximum(bx1S, bx1L)
        my1 = jnp.maximum(by1S, by1L)
        mx2 = jnp.minimum(bx2S, bx2L)
        my2 = jnp.minimum(by2S, by2L)
        mw = jnp.maximum(mx2 - mx1, 0.0)
        mh = jnp.maximum(my2 - my1, 0.0)
        minter = mw * mh
        miou = minter / (areaS + areaL - minter + 1e-6)
        mhit = jnp.logical_and(miou >= _THR, sub2 < lane2)

        kl_ref[...] = alive0.astype(jnp.int32)

        def fix_body(_):
            kl = kl_ref[...] != 0
            ks = jnp.any(jnp.logical_and(lane2 == sub2, kl), axis=1,
                         keepdims=True)
            sup = jnp.any(jnp.logical_and(mhit, ks), axis=0,
                          keepdims=True)
            knew = jnp.logical_and(alive0, jnp.logical_not(sup))
            kl_ref[...] = knew.astype(jnp.int32)
            return jnp.any(knew != kl)

        lax.while_loop(lambda c: c, fix_body, True)
        keepL = kl_ref[...] != 0
        keepf = keepL.astype(jnp.float32)

        keepS = jnp.any(jnp.logical_and(lane2 == sub2, keepL), axis=1,
                        keepdims=True)
        base = cidx * _LANES
        kx1_ref[pl.ds(base, _LANES), :] = jnp.broadcast_to(
            jnp.where(keepS, bx1S, -1e9), (_LANES, _LANES))
        ky1_ref[pl.ds(base, _LANES), :] = jnp.broadcast_to(
            jnp.where(keepS, by1S, -1e9), (_LANES, _LANES))
        kx2_ref[pl.ds(base, _LANES), :] = jnp.broadcast_to(
            jnp.where(keepS, bx2S, -1e9), (_LANES, _LANES))
        ky2_ref[pl.ds(base, _LANES), :] = jnp.broadcast_to(
            jnp.where(keepS, by2S, -1e9), (_LANES, _LANES))
        karea_ref[pl.ds(base, _LANES), :] = jnp.broadcast_to(
            jnp.where(keepS, areaS, 0.0), (_LANES, _LANES))

        prefixL = lax.dot_general(keepf, ltri, (((1,), (0,)), ((), ())),
                                  precision=lax.Precision.HIGHEST)
        pmat = jnp.logical_and(sub2 == prefixL.astype(jnp.int32),
                               keepL).astype(jnp.float32)
        vmat = jnp.concatenate([bx1S, by1S, bx2S, by2S, bsS,
                                jnp.zeros((_LANES, 3), jnp.float32)],
                               axis=1)
        compact = lax.dot_general(pmat, vmat, (((1,), (0,)), ((), ())),
                                  precision=lax.Precision.HIGHEST)
        oacc_ref[pl.ds(count, _LANES), :] = compact

        nkeep = jnp.sum(keepf).astype(jnp.int32)
        return (cidx + 1, count + nkeep)

    _, count = lax.while_loop(cond, body, (jnp.int32(0), jnp.int32(0)))
    out_ref[...] = oacc_ref[0:_MAX_OUT, 0:5]
    flag_ref[...] = jnp.reshape((count < _MAX_OUT).astype(jnp.int32), (1, 1))


def _run_nms_chunked(planes, subplanes):
    return pl.pallas_call(
        _nms_chunked_body,
        out_shape=(
            jax.ShapeDtypeStruct((_MAX_OUT, 5), jnp.float32),
            jax.ShapeDtypeStruct((1, 1), jnp.int32),
        ),
        scratch_shapes=[pltpu.VMEM((_SLOTS, _LANES), jnp.float32)] * 5
        + [pltpu.VMEM((_SLOTS, 8), jnp.float32),
           pltpu.VMEM((1, _LANES), jnp.int32)],
    )(planes, subplanes)


def _nms_body(nlimit, planes_ref, out_ref, flag_ref,
              kx1_ref, ky1_ref, kx2_ref, ky2_ref, karea_ref):
    out_ref[...] = jnp.zeros((_MAX_OUT, 5), jnp.float32)
    kx1_ref[...] = jnp.zeros((_KSLOT, _LANES), jnp.float32)
    ky1_ref[...] = jnp.zeros((_KSLOT, _LANES), jnp.float32)
    kx2_ref[...] = jnp.zeros((_KSLOT, _LANES), jnp.float32)
    ky2_ref[...] = jnp.zeros((_KSLOT, _LANES), jnp.float32)
    karea_ref[...] = jnp.zeros((_KSLOT, _LANES), jnp.float32)

    lane_iota = lax.broadcasted_iota(jnp.int32, (1, 1, _LANES), 2)
    slot_rows = lax.broadcasted_iota(jnp.int32, (_KSLOT, _LANES), 0)
    slot_lanes = lax.broadcasted_iota(jnp.int32, (_KSLOT, _LANES), 1)
    slot_iota = slot_rows * _LANES + slot_lanes

    def cond(state):
        p, count = state
        return jnp.logical_and(count < _MAX_OUT, p < nlimit)

    def body(state):
        p, count = state
        r = p // _LANES
        c = p - r * _LANES
        blk = planes_ref[:, pl.ds(r, 1), :]
        sel = jnp.sum(jnp.where(lane_iota == c, blk, 0.0), axis=2)
        bx1 = sel[0:1, :]
        by1 = sel[1:2, :]
        bx2 = sel[2:3, :]
        by2 = sel[3:4, :]
        bs = sel[4:5, :]

        xx1 = jnp.maximum(kx1_ref[...], bx1)
        yy1 = jnp.maximum(ky1_ref[...], by1)
        xx2 = jnp.minimum(kx2_ref[...], bx2)
        yy2 = jnp.minimum(ky2_ref[...], by2)
        w = jnp.maximum(xx2 - xx1, 0.0)
        h = jnp.maximum(yy2 - yy1, 0.0)
        inter = w * h
        area_a = (bx2 - bx1) * (by2 - by1)
        iou = inter / (area_a + karea_ref[...] - inter + 1e-6)
        hit = jnp.logical_and(iou >= _THR, slot_iota < count)
        keep = jnp.logical_not(jnp.any(hit))

        @pl.when(keep)
        def _():
            onehot = slot_iota == count
            kx1_ref[...] = jnp.where(onehot, bx1, kx1_ref[...])
            ky1_ref[...] = jnp.where(onehot, by1, ky1_ref[...])
            kx2_ref[...] = jnp.where(onehot, bx2, kx2_ref[...])
            ky2_ref[...] = jnp.where(onehot, by2, ky2_ref[...])
            karea_ref[...] = jnp.where(onehot, area_a, karea_ref[...])
            out_ref[pl.ds(count, 1), 0:1] = bx1
            out_ref[pl.ds(count, 1), 1:2] = by1
            out_ref[pl.ds(count, 1), 2:3] = bx2
            out_ref[pl.ds(count, 1), 3:4] = by2
            out_ref[pl.ds(count, 1), 4:5] = bs

        return (p + 1, count + keep.astype(jnp.int32))

    _, count = lax.while_loop(cond, body, (jnp.int32(0), jnp.int32(0)))
    flag_ref[...] = jnp.reshape((count < _MAX_OUT).astype(jnp.int32), (1, 1))


def _run_nms(planes, nlimit):
    return pl.pallas_call(
        functools.partial(_nms_body, nlimit),
        out_shape=(
            jax.ShapeDtypeStruct((_MAX_OUT, 5), jnp.float32),
            jax.ShapeDtypeStruct((1, 1), jnp.int32),
        ),
        scratch_shapes=[pltpu.VMEM((_KSLOT, _LANES), jnp.float32)] * 5,
    )(planes)


def _make_planes(sb, ss, npad):
    cols = jnp.concatenate([sb, ss[:, None]], axis=1)
    cols = jnp.pad(cols, ((0, npad - cols.shape[0]), (0, 0)))
    return cols.T.reshape(5, npad // _LANES, _LANES)


def kernel(boxes, scores):
    ss, order = lax.top_k(scores, _K)
    sb = jnp.take(boxes, order, axis=0)
    planes = _make_planes(sb, ss, _K)
    subplanes = jnp.concatenate([sb, ss[:, None]], axis=1).T.reshape(5, _K, 1)
    out_fast, flag = _run_nms_chunked(planes, subplanes)

    def full_path(_):
        order_f = jnp.argsort(-scores)
        sb_f = jnp.take(boxes, order_f, axis=0)
        ss_f = jnp.take(scores, order_f, axis=0)
        out_full, _unused = _run_nms(_make_planes(sb_f, ss_f, _PAD), _N)
        return out_full

    return lax.cond(flag[0, 0] > 0, full_path, lambda _: out_fast, None)


# speedup vs baseline: 4.6071x; 1.0241x over previous
import functools

import jax
import jax.numpy as jnp
from jax import lax
from jax.experimental import pallas as pl
from jax.experimental.pallas import tpu as pltpu

_N = 20000
_PAD = 20480
_K = 640
_NCHUNK = _K // 128
_MAX_OUT = 500
_KSLOT = 4
_SLOTS = 640
_LANES = 128
_THR = 0.5


def _nms_chunked_body(planes_ref, sub_ref, out_ref, flag_ref,
                      kx1_ref, ky1_ref, kx2_ref, ky2_ref, karea_ref,
                      oacc_ref, kl_ref):
    out_ref[...] = jnp.zeros((_MAX_OUT, 5), jnp.float32)
    kx1_ref[...] = jnp.full((_SLOTS, _LANES), -1e9, jnp.float32)
    ky1_ref[...] = jnp.full((_SLOTS, _LANES), -1e9, jnp.float32)
    kx2_ref[...] = jnp.full((_SLOTS, _LANES), -1e9, jnp.float32)
    ky2_ref[...] = jnp.full((_SLOTS, _LANES), -1e9, jnp.float32)
    karea_ref[...] = jnp.zeros((_SLOTS, _LANES), jnp.float32)
    oacc_ref[...] = jnp.zeros((_SLOTS, 8), jnp.float32)

    sub2 = lax.broadcasted_iota(jnp.int32, (_LANES, _LANES), 0)
    lane2 = lax.broadcasted_iota(jnp.int32, (_LANES, _LANES), 1)
    ltri = (sub2 < lane2).astype(jnp.float32)

    count = jnp.int32(0)
    for cidx in range(_NCHUNK):
        active = count < _MAX_OUT

        def getL(i, c=cidx):
            return jnp.reshape(planes_ref[i:i + 1, c:c + 1, :], (1, _LANES))

        def getS(i, c=cidx):
            return jnp.reshape(
                sub_ref[i:i + 1, c * _LANES:(c + 1) * _LANES, :], (_LANES, 1))

        bx1L, by1L, bx2L, by2L = getL(0), getL(1), getL(2), getL(3)
        bx1S, by1S, bx2S, by2S, bsS = getS(0), getS(1), getS(2), getS(3), getS(4)
        areaL = (bx2L - bx1L) * (by2L - by1L)
        areaS = (bx2S - bx1S) * (by2S - by1S)

        if cidx == 0:
            alive0 = jnp.ones((1, _LANES), jnp.bool_)
        else:
            nsl = cidx * _LANES
            xx1 = jnp.maximum(kx1_ref[0:nsl, :], bx1L)
            yy1 = jnp.maximum(ky1_ref[0:nsl, :], by1L)
            xx2 = jnp.minimum(kx2_ref[0:nsl, :], bx2L)
            yy2 = jnp.minimum(ky2_ref[0:nsl, :], by2L)
            w = jnp.maximum(xx2 - xx1, 0.0)
            h = jnp.maximum(yy2 - yy1, 0.0)
            inter = w * h
            iou = inter / (areaL + karea_ref[0:nsl, :] - inter + 1e-6)
            alive0 = jnp.logical_not(
                jnp.any(iou >= _THR, axis=0, keepdims=True))

        mx1 = jnp.maximum(bx1S, bx1L)
        my1 = jnp.maximum(by1S, by1L)
        mx2 = jnp.minimum(bx2S, bx2L)
        my2 = jnp.minimum(by2S, by2L)
        mw = jnp.maximum(mx2 - mx1, 0.0)
        mh = jnp.maximum(my2 - my1, 0.0)
        minter = mw * mh
        miou = minter / (areaS + areaL - minter + 1e-6)
        mhit = jnp.logical_and(miou >= _THR, sub2 < lane2)

        kl_ref[...] = alive0.astype(jnp.int32)

        def fix_body(_):
            kl = kl_ref[...] != 0
            ks = jnp.any(jnp.logical_and(lane2 == sub2, kl), axis=1,
                         keepdims=True)
            sup = jnp.any(jnp.logical_and(mhit, ks), axis=0,
                          keepdims=True)
            knew = jnp.logical_and(alive0, jnp.logical_not(sup))
            kl_ref[...] = knew.astype(jnp.int32)
            return jnp.any(knew != kl)

        lax.while_loop(lambda c: c, fix_body, True)
        keepL = kl_ref[...] != 0
        keepf = keepL.astype(jnp.float32)

        keepS = jnp.any(jnp.logical_and(lane2 == sub2, keepL), axis=1,
                        keepdims=True)

        prefixL = lax.dot_general(keepf, ltri, (((1,), (0,)), ((), ())),
                                  precision=lax.Precision.DEFAULT)
        pmat = jnp.logical_and(sub2 == prefixL.astype(jnp.int32),
                               keepL).astype(jnp.float32)
        vmat = jnp.concatenate([bx1S, by1S, bx2S, by2S, bsS,
                                jnp.zeros((_LANES, 3), jnp.float32)],
                               axis=1)
        compact = lax.dot_general(pmat, vmat, (((1,), (0,)), ((), ())),
                                  precision=lax.Precision.HIGHEST)

        @pl.when(active)
        def _(c=cidx, keepS=keepS, bx1S=bx1S, by1S=by1S, bx2S=bx2S,
              by2S=by2S, areaS=areaS, compact=compact, count=count):
            base = c * _LANES
            kx1_ref[base:base + _LANES, :] = jnp.broadcast_to(
                jnp.where(keepS, bx1S, -1e9), (_LANES, _LANES))
            ky1_ref[base:base + _LANES, :] = jnp.broadcast_to(
                jnp.where(keepS, by1S, -1e9), (_LANES, _LANES))
            kx2_ref[base:base + _LANES, :] = jnp.broadcast_to(
                jnp.where(keepS, bx2S, -1e9), (_LANES, _LANES))
            ky2_ref[base:base + _LANES, :] = jnp.broadcast_to(
                jnp.where(keepS, by2S, -1e9), (_LANES, _LANES))
            karea_ref[base:base + _LANES, :] = jnp.broadcast_to(
                jnp.where(keepS, areaS, 0.0), (_LANES, _LANES))
            oacc_ref[pl.ds(count, _LANES), :] = compact

        nkeep = jnp.sum(keepf).astype(jnp.int32)
        count = jnp.where(active, count + nkeep, count)

    out_ref[...] = oacc_ref[0:_MAX_OUT, 0:5]
    flag_ref[...] = jnp.reshape((count < _MAX_OUT).astype(jnp.int32), (1, 1))


def _run_nms_chunked(planes, subplanes):
    return pl.pallas_call(
        _nms_chunked_body,
        out_shape=(
            jax.ShapeDtypeStruct((_MAX_OUT, 5), jnp.float32),
            jax.ShapeDtypeStruct((1, 1), jnp.int32),
        ),
        scratch_shapes=[pltpu.VMEM((_SLOTS, _LANES), jnp.float32)] * 5
        + [pltpu.VMEM((_SLOTS, 8), jnp.float32),
           pltpu.VMEM((1, _LANES), jnp.int32)],
    )(planes, subplanes)


def _nms_body(nlimit, planes_ref, out_ref, flag_ref,
              kx1_ref, ky1_ref, kx2_ref, ky2_ref, karea_ref):
    out_ref[...] = jnp.zeros((_MAX_OUT, 5), jnp.float32)
    kx1_ref[...] = jnp.zeros((_KSLOT, _LANES), jnp.float32)
    ky1_ref[...] = jnp.zeros((_KSLOT, _LANES), jnp.float32)
    kx2_ref[...] = jnp.zeros((_KSLOT, _LANES), jnp.float32)
    ky2_ref[...] = jnp.zeros((_KSLOT, _LANES), jnp.float32)
    karea_ref[...] = jnp.zeros((_KSLOT, _LANES), jnp.float32)

    lane_iota = lax.broadcasted_iota(jnp.int32, (1, 1, _LANES), 2)
    slot_rows = lax.broadcasted_iota(jnp.int32, (_KSLOT, _LANES), 0)
    slot_lanes = lax.broadcasted_iota(jnp.int32, (_KSLOT, _LANES), 1)
    slot_iota = slot_rows * _LANES + slot_lanes

    def cond(state):
        p, count = state
        return jnp.logical_and(count < _MAX_OUT, p < nlimit)

    def body(state):
        p, count = state
        r = p // _LANES
        c = p - r * _LANES
        blk = planes_ref[:, pl.ds(r, 1), :]
        sel = jnp.sum(jnp.where(lane_iota == c, blk, 0.0), axis=2)
        bx1 = sel[0:1, :]
        by1 = sel[1:2, :]
        bx2 = sel[2:3, :]
        by2 = sel[3:4, :]
        bs = sel[4:5, :]

        xx1 = jnp.maximum(kx1_ref[...], bx1)
        yy1 = jnp.maximum(ky1_ref[...], by1)
        xx2 = jnp.minimum(kx2_ref[...], bx2)
        yy2 = jnp.minimum(ky2_ref[...], by2)
        w = jnp.maximum(xx2 - xx1, 0.0)
        h = jnp.maximum(yy2 - yy1, 0.0)
        inter = w * h
        area_a = (bx2 - bx1) * (by2 - by1)
        iou = inter / (area_a + karea_ref[...] - inter + 1e-6)
        hit = jnp.logical_and(iou >= _THR, slot_iota < count)
        keep = jnp.logical_not(jnp.any(hit))

        @pl.when(keep)
        def _():
            onehot = slot_iota == count
            kx1_ref[...] = jnp.where(onehot, bx1, kx1_ref[...])
            ky1_ref[...] = jnp.where(onehot, by1, ky1_ref[...])
            kx2_ref[...] = jnp.where(onehot, bx2, kx2_ref[...])
            ky2_ref[...] = jnp.where(onehot, by2, ky2_ref[...])
            karea_ref[...] = jnp.where(onehot, area_a, karea_ref[...])
            out_ref[pl.ds(count, 1), 0:1] = bx1
            out_ref[pl.ds(count, 1), 1:2] = by1
            out_ref[pl.ds(count, 1), 2:3] = bx2
            out_ref[pl.ds(count, 1), 3:4] = by2
            out_ref[pl.ds(count, 1), 4:5] = bs

        return (p + 1, count + keep.astype(jnp.int32))

    _, count = lax.while_loop(cond, body, (jnp.int32(0), jnp.int32(0)))
    flag_ref[...] = jnp.reshape((count < _MAX_OUT).astype(jnp.int32), (1, 1))


def _run_nms(planes, nlimit):
    return pl.pallas_call(
        functools.partial(_nms_body, nlimit),
        out_shape=(
            jax.ShapeDtypeStruct((_MAX_OUT, 5), jnp.float32),
            jax.ShapeDtypeStruct((1, 1), jnp.int32),
        ),
        scratch_shapes=[pltpu.VMEM((_KSLOT, _LANES), jnp.float32)] * 5,
    )(planes)


def _make_planes(sb, ss, npad):
    cols = jnp.concatenate([sb, ss[:, None]], axis=1)
    cols = jnp.pad(cols, ((0, npad - cols.shape[0]), (0, 0)))
    return cols.T.reshape(5, npad // _LANES, _LANES)


def kernel(boxes, scores):
    ss, order = lax.top_k(scores, _K)
    sb = jnp.take(boxes, order, axis=0)
    planes = _make_planes(sb, ss, _K)
    subplanes = jnp.concatenate([sb, ss[:, None]], axis=1).T.reshape(5, _K, 1)
    out_fast, flag = _run_nms_chunked(planes, subplanes)

    def full_path(_):
        order_f = jnp.argsort(-scores)
        sb_f = jnp.take(boxes, order_f, axis=0)
        ss_f = jnp.take(scores, order_f, axis=0)
        out_full, _unused = _run_nms(_make_planes(sb_f, ss_f, _PAD), _N)
        return out_full

    return lax.cond(flag[0, 0] > 0, full_path, lambda _: out_fast, None)


# speedup vs baseline: 4.7014x; 1.0205x over previous
import functools

import jax
import jax.numpy as jnp
from jax import lax
from jax.experimental import pallas as pl
from jax.experimental.pallas import tpu as pltpu

_N = 20000
_PAD = 20480
_K = 640
_NCHUNK = _K // 128
_MAX_OUT = 500
_KSLOT = 4
_SLOTS = 640
_LANES = 128
_THR = 0.5


def _nms_chunked_body(rows_ref, out_ref, flag_ref,
                      kx1_ref, ky1_ref, kx2_ref, ky2_ref, karea_ref,
                      oacc_ref, kl_ref):
    out_ref[...] = jnp.zeros((_MAX_OUT, 5), jnp.float32)
    kx1_ref[...] = jnp.full((_SLOTS, _LANES), -1e9, jnp.float32)
    ky1_ref[...] = jnp.full((_SLOTS, _LANES), -1e9, jnp.float32)
    kx2_ref[...] = jnp.full((_SLOTS, _LANES), -1e9, jnp.float32)
    ky2_ref[...] = jnp.full((_SLOTS, _LANES), -1e9, jnp.float32)
    karea_ref[...] = jnp.zeros((_SLOTS, _LANES), jnp.float32)
    oacc_ref[...] = jnp.zeros((_SLOTS, 8), jnp.float32)

    sub2 = lax.broadcasted_iota(jnp.int32, (_LANES, _LANES), 0)
    lane2 = lax.broadcasted_iota(jnp.int32, (_LANES, _LANES), 1)
    ltri = (sub2 < lane2).astype(jnp.float32)

    count = jnp.int32(0)
    for cidx in range(_NCHUNK):
        active = count < _MAX_OUT

        def getS(i, c=cidx):
            return rows_ref[c * _LANES:(c + 1) * _LANES, i:i + 1]

        def tr(vS):
            return jnp.sum(jnp.where(sub2 == lane2, vS, 0.0), axis=0,
                           keepdims=True)

        bx1S, by1S, bx2S, by2S, bsS = getS(0), getS(1), getS(2), getS(3), getS(4)
        areaS = (bx2S - bx1S) * (by2S - by1S)
        bx1L, by1L, bx2L, by2L = tr(bx1S), tr(by1S), tr(bx2S), tr(by2S)
        areaL = tr(areaS)

        if cidx == 0:
            alive0 = jnp.ones((1, _LANES), jnp.bool_)
        else:
            nsl = cidx * _LANES
            xx1 = jnp.maximum(kx1_ref[0:nsl, :], bx1L)
            yy1 = jnp.maximum(ky1_ref[0:nsl, :], by1L)
            xx2 = jnp.minimum(kx2_ref[0:nsl, :], bx2L)
            yy2 = jnp.minimum(ky2_ref[0:nsl, :], by2L)
            w = jnp.maximum(xx2 - xx1, 0.0)
            h = jnp.maximum(yy2 - yy1, 0.0)
            inter = w * h
            iou = inter / (areaL + karea_ref[0:nsl, :] - inter + 1e-6)
            alive0 = jnp.logical_not(
                jnp.any(iou >= _THR, axis=0, keepdims=True))

        mx1 = jnp.maximum(bx1S, bx1L)
        my1 = jnp.maximum(by1S, by1L)
        mx2 = jnp.minimum(bx2S, bx2L)
        my2 = jnp.minimum(by2S, by2L)
        mw = jnp.maximum(mx2 - mx1, 0.0)
        mh = jnp.maximum(my2 - my1, 0.0)
        minter = mw * mh
        miou = minter / (areaS + areaL - minter + 1e-6)
        mhit = jnp.logical_and(miou >= _THR, sub2 < lane2)

        kl_ref[...] = alive0.astype(jnp.int32)

        def fix_body(_):
            kl = kl_ref[...] != 0
            ks = jnp.any(jnp.logical_and(lane2 == sub2, kl), axis=1,
                         keepdims=True)
            sup = jnp.any(jnp.logical_and(mhit, ks), axis=0,
                          keepdims=True)
            knew = jnp.logical_and(alive0, jnp.logical_not(sup))
            kl_ref[...] = knew.astype(jnp.int32)
            return jnp.any(knew != kl)

        lax.while_loop(lambda c: c, fix_body, True)
        keepL = kl_ref[...] != 0
        keepf = keepL.astype(jnp.float32)

        keepS = jnp.any(jnp.logical_and(lane2 == sub2, keepL), axis=1,
                        keepdims=True)

        prefixL = lax.dot_general(keepf, ltri, (((1,), (0,)), ((), ())),
                                  precision=lax.Precision.DEFAULT)
        pmat = jnp.logical_and(sub2 == prefixL.astype(jnp.int32),
                               keepL).astype(jnp.float32)
        vmat = jnp.concatenate([bx1S, by1S, bx2S, by2S, bsS,
                                jnp.zeros((_LANES, 3), jnp.float32)],
                               axis=1)
        compact = lax.dot_general(pmat, vmat, (((1,), (0,)), ((), ())),
                                  precision=lax.Precision.HIGHEST)

        @pl.when(active)
        def _(c=cidx, keepS=keepS, bx1S=bx1S, by1S=by1S, bx2S=bx2S,
              by2S=by2S, areaS=areaS, compact=compact, count=count):
            base = c * _LANES
            kx1_ref[base:base + _LANES, :] = jnp.broadcast_to(
                jnp.where(keepS, bx1S, -1e9), (_LANES, _LANES))
            ky1_ref[base:base + _LANES, :] = jnp.broadcast_to(
                jnp.where(keepS, by1S, -1e9), (_LANES, _LANES))
            kx2_ref[base:base + _LANES, :] = jnp.broadcast_to(
                jnp.where(keepS, bx2S, -1e9), (_LANES, _LANES))
            ky2_ref[base:base + _LANES, :] = jnp.broadcast_to(
                jnp.where(keepS, by2S, -1e9), (_LANES, _LANES))
            karea_ref[base:base + _LANES, :] = jnp.broadcast_to(
                jnp.where(keepS, areaS, 0.0), (_LANES, _LANES))
            oacc_ref[pl.ds(count, _LANES), :] = compact

        nkeep = jnp.sum(keepf).astype(jnp.int32)
        count = jnp.where(active, count + nkeep, count)

    out_ref[...] = oacc_ref[0:_MAX_OUT, 0:5]
    flag_ref[...] = jnp.reshape((count < _MAX_OUT).astype(jnp.int32), (1, 1))


def _run_nms_chunked(rows):
    return pl.pallas_call(
        _nms_chunked_body,
        out_shape=(
            jax.ShapeDtypeStruct((_MAX_OUT, 5), jnp.float32),
            jax.ShapeDtypeStruct((1, 1), jnp.int32),
        ),
        scratch_shapes=[pltpu.VMEM((_SLOTS, _LANES), jnp.float32)] * 5
        + [pltpu.VMEM((_SLOTS, 8), jnp.float32),
           pltpu.VMEM((1, _LANES), jnp.int32)],
    )(rows)


def _nms_body(nlimit, planes_ref, out_ref, flag_ref,
              kx1_ref, ky1_ref, kx2_ref, ky2_ref, karea_ref):
    out_ref[...] = jnp.zeros((_MAX_OUT, 5), jnp.float32)
    kx1_ref[...] = jnp.zeros((_KSLOT, _LANES), jnp.float32)
    ky1_ref[...] = jnp.zeros((_KSLOT, _LANES), jnp.float32)
    kx2_ref[...] = jnp.zeros((_KSLOT, _LANES), jnp.float32)
    ky2_ref[...] = jnp.zeros((_KSLOT, _LANES), jnp.float32)
    karea_ref[...] = jnp.zeros((_KSLOT, _LANES), jnp.float32)

    lane_iota = lax.broadcasted_iota(jnp.int32, (1, 1, _LANES), 2)
    slot_rows = lax.broadcasted_iota(jnp.int32, (_KSLOT, _LANES), 0)
    slot_lanes = lax.broadcasted_iota(jnp.int32, (_KSLOT, _LANES), 1)
    slot_iota = slot_rows * _LANES + slot_lanes

    def cond(state):
        p, count = state
        return jnp.logical_and(count < _MAX_OUT, p < nlimit)

    def body(state):
        p, count = state
        r = p // _LANES
        c = p - r * _LANES
        blk = planes_ref[:, pl.ds(r, 1), :]
        sel = jnp.sum(jnp.where(lane_iota == c, blk, 0.0), axis=2)
        bx1 = sel[0:1, :]
        by1 = sel[1:2, :]
        bx2 = sel[2:3, :]
        by2 = sel[3:4, :]
        bs = sel[4:5, :]

        xx1 = jnp.maximum(kx1_ref[...], bx1)
        yy1 = jnp.maximum(ky1_ref[...], by1)
        xx2 = jnp.minimum(kx2_ref[...], bx2)
        yy2 = jnp.minimum(ky2_ref[...], by2)
        w = jnp.maximum(xx2 - xx1, 0.0)
        h = jnp.maximum(yy2 - yy1, 0.0)
        inter = w * h
        area_a = (bx2 - bx1) * (by2 - by1)
        iou = inter / (area_a + karea_ref[...] - inter + 1e-6)
        hit = jnp.logical_and(iou >= _THR, slot_iota < count)
        keep = jnp.logical_not(jnp.any(hit))

        @pl.when(keep)
        def _():
            onehot = slot_iota == count
            kx1_ref[...] = jnp.where(onehot, bx1, kx1_ref[...])
            ky1_ref[...] = jnp.where(onehot, by1, ky1_ref[...])
            kx2_ref[...] = jnp.where(onehot, bx2, kx2_ref[...])
            ky2_ref[...] = jnp.where(onehot, by2, ky2_ref[...])
            karea_ref[...] = jnp.where(onehot, area_a, karea_ref[...])
            out_ref[pl.ds(count, 1), 0:1] = bx1
            out_ref[pl.ds(count, 1), 1:2] = by1
            out_ref[pl.ds(count, 1), 2:3] = bx2
            out_ref[pl.ds(count, 1), 3:4] = by2
            out_ref[pl.ds(count, 1), 4:5] = bs

        return (p + 1, count + keep.astype(jnp.int32))

    _, count = lax.while_loop(cond, body, (jnp.int32(0), jnp.int32(0)))
    flag_ref[...] = jnp.reshape((count < _MAX_OUT).astype(jnp.int32), (1, 1))


def _run_nms(planes, nlimit):
    return pl.pallas_call(
        functools.partial(_nms_body, nlimit),
        out_shape=(
            jax.ShapeDtypeStruct((_MAX_OUT, 5), jnp.float32),
            jax.ShapeDtypeStruct((1, 1), jnp.int32),
        ),
        scratch_shapes=[pltpu.VMEM((_KSLOT, _LANES), jnp.float32)] * 5,
    )(planes)


def _make_planes(sb, ss, npad):
    cols = jnp.concatenate([sb, ss[:, None]], axis=1)
    cols = jnp.pad(cols, ((0, npad - cols.shape[0]), (0, 0)))
    return cols.T.reshape(5, npad // _LANES, _LANES)


def kernel(boxes, scores):
    ss, order = lax.top_k(scores, _K)
    sb = jnp.take(boxes, order, axis=0)
    rows = jnp.concatenate([sb, ss[:, None]], axis=1)
    out_fast, flag = _run_nms_chunked(rows)

    def full_path(_):
        order_f = jnp.argsort(-scores)
        sb_f = jnp.take(boxes, order_f, axis=0)
        ss_f = jnp.take(scores, order_f, axis=0)
        out_full, _unused = _run_nms(_make_planes(sb_f, ss_f, _PAD), _N)
        return out_full

    return lax.cond(flag[0, 0] > 0, full_path, lambda _: out_fast, None)
